# baseline (speedup 1.0000x reference)
import jax
import jax.numpy as jnp
from jax import lax
from jax.experimental import pallas as pl
from jax.experimental.pallas import tpu as pltpu
from jax.experimental.pallas import tpu_sc as plsc

N = 10000
D = 128
H = 256
E = 320000

NC = 2
NS = 16
B = 40
EPT = E // NS
NBATCH = EPT // B
NP = NBATCH // 2
WB = 624


def _table_body(nodes_ref, w1_ref, b1_ref, out_ref):
    n = nodes_ref[...]
    w = w1_ref[...]
    pa = lax.dot_general(n, w[:, :D], (((1,), (1,)), ((), ())),
                         preferred_element_type=jnp.float32)
    pb = lax.dot_general(n, w[:, D:], (((1,), (1,)), ((), ())),
                         preferred_element_type=jnp.float32)
    out_ref[0, :, :D] = pa + b1_ref[0, 0:1, :]
    out_ref[0, :, D:] = pb


def _build_table(nodes, W1, b1):
    nb = 10
    blk = N // nb
    return pl.pallas_call(
        _table_body,
        grid=(NC, nb),
        in_specs=[
            pl.BlockSpec((blk, D), lambda c, b: (b, 0)),
            pl.BlockSpec((H // NC, 2 * D), lambda c, b: (c, 0)),
            pl.BlockSpec((1, 1, H // NC), lambda c, b: (c, 0, 0)),
        ],
        out_specs=pl.BlockSpec((1, blk, 2 * D), lambda c, b: (c, b, 0)),
        out_shape=jax.ShapeDtypeStruct((NC, N, 2 * D), jnp.float32),
    )(nodes, W1, b1.reshape(NC, 1, H // NC))


def _sc_edge_body(t0, t1, ivec, jvec, u2,
                  acc_out,
                  idx_ia, idx_ja, idx_ib, idx_jb,
                  rows_ia, rows_ja, rows_ib, rows_jb,
                  h_v, u_v,
                  acc_sh,
                  sga1, sga2, sgb1, sgb2, ssc, ssc2, si1, si2):
    c = lax.axis_index("c")
    s = lax.axis_index("s")

    def zrow(r, _):
        for v in range(D // 16):
            h_v[r, pl.ds(v * 16, 16)] = jnp.zeros((16,), jnp.float32)
        return 0
    lax.fori_loop(0, B, zrow, 0)
    for k in range(15):
        pltpu.sync_copy(h_v, acc_sh.at[pl.ds(s * (N // NS) + k * B, B)])
    pltpu.sync_copy(h_v.at[pl.ds(0, 25)],
                    acc_sh.at[pl.ds(s * (N // NS) + 15 * B, 25)])
    plsc.subcore_barrier()

    pltpu.sync_copy(u2.at[c], u_v)
    uv = [u_v[0, pl.ds(v * 16, 16)] for v in range(D // 16)]

    def load_idx(t, ii, jj):
        base = s * EPT + t * B
        cpi = pltpu.make_async_copy(ivec.at[pl.ds(base, B)], ii.at[0], si1)
        cpj = pltpu.make_async_copy(jvec.at[pl.ds(base, B)], jj.at[0], si2)
        cpi.start()
        cpj.start()
        cpi.wait()
        cpj.wait()

    NSPL = 5
    SB = B // NSPL

    def _gather_parts(tref, ii, jj, ri, rj, sg1, sg2):
        parts = []
        for k in range(NSPL):
            ksl = pl.ds(k * SB, SB)
            parts.append(pltpu.make_async_copy(
                tref.at[ii.at[0, ksl]], ri.at[ksl], sg1))
            parts.append(pltpu.make_async_copy(
                tref.at[jj.at[0, ksl]], rj.at[ksl], sg2))
        return parts

    def start_gathers(ii, jj, ri, rj, sg1, sg2):
        @pl.when(c == 0)
        def _():
            for cp in _gather_parts(t0, ii, jj, ri, rj, sg1, sg2):
                cp.start()

        @pl.when(c == 1)
        def _():
            for cp in _gather_parts(t1, ii, jj, ri, rj, sg1, sg2):
                cp.start()

    def wait_gathers(ii, jj, ri, rj, sg1, sg2):
        @pl.when(c == 0)
        def _():
            for cp in _gather_parts(t0, ii, jj, ri, rj, sg1, sg2):
                cp.wait()

        @pl.when(c == 1)
        def _():
            for cp in _gather_parts(t1, ii, jj, ri, rj, sg1, sg2):
                cp.wait()

    def compute_h(rx, ry, hbuf):
        def eb(b2, _):
            b = 2 * b2
            for e in range(2):
                for v in range(D // 16):
                    sl = pl.ds(v * 16, 16)
                    sh = pl.ds(D + v * 16, 16)
                    hbuf[b + e, sl] = jnp.maximum(
                        rx[b + e, sl] + ry[b + e, sh], 0.0) + uv[v]
            return 0
        lax.fori_loop(0, B // 2, eb, 0)

    def _scatter_parts(hbuf, idxr, sem):
        parts = []
        for k in range(NSPL):
            ksl = pl.ds(k * SB, SB)
            parts.append(pltpu.make_async_copy(
                hbuf.at[ksl], acc_sh.at[idxr.at[0, ksl]], sem))
        return parts

    def consume(ii, jj, ri, rj, sg1, sg2):
        wait_gathers(ii, jj, ri, rj, sg1, sg2)
        compute_h(ri, rj, h_v)
        p1 = _scatter_parts(h_v, ii, ssc)
        for cp in p1:
            cp.start(add=True)
        for cp in p1:
            cp.wait()
        compute_h(rj, ri, h_v)
        p2 = _scatter_parts(h_v, jj, ssc2)
        for cp in p2:
            cp.start(add=True)
        for cp in p2:
            cp.wait()

    load_idx(0, idx_ia, idx_ja)
    start_gathers(idx_ia, idx_ja, rows_ia, rows_ja, sga1, sga2)

    def pair_body(p, _):
        load_idx(2 * p + 1, idx_ib, idx_jb)
        start_gathers(idx_ib, idx_jb, rows_ib, rows_jb, sgb1, sgb2)
        consume(idx_ia, idx_ja, rows_ia, rows_ja, sga1, sga2)

        @pl.when(p < NP - 1)
        def _():
            load_idx(2 * p + 2, idx_ia, idx_ja)
            start_gathers(idx_ia, idx_ja, rows_ia, rows_ja, sga1, sga2)
        consume(idx_ib, idx_jb, rows_ib, rows_jb, sgb1, sgb2)
        return 0
    lax.fori_loop(0, NP, pair_body, 0)

    plsc.subcore_barrier()

    rsl = pl.ds(s * WB, WB)
    pltpu.sync_copy(acc_sh.at[rsl], acc_out.at[c, rsl])

    @pl.when(s == NS - 1)
    def _():
        tsl = pl.ds(NS * WB, N - NS * WB)
        pltpu.sync_copy(acc_sh.at[tsl], acc_out.at[c, tsl])


def _sc_edge_accumulate(t0, t1, ivec, jvec, u2):
    mesh = plsc.VectorSubcoreMesh(core_axis_name="c", subcore_axis_name="s",
                                  num_cores=NC, num_subcores=NS)
    f = pl.kernel(
        _sc_edge_body,
        out_type=jax.ShapeDtypeStruct((NC, N, D), jnp.float32),
        mesh=mesh,
        scratch_types=[
            pltpu.VMEM((1, B), jnp.int32),
            pltpu.VMEM((1, B), jnp.int32),
            pltpu.VMEM((1, B), jnp.int32),
            pltpu.VMEM((1, B), jnp.int32),
            pltpu.VMEM((B, 2 * D), jnp.float32),
            pltpu.VMEM((B, 2 * D), jnp.float32),
            pltpu.VMEM((B, 2 * D), jnp.float32),
            pltpu.VMEM((B, 2 * D), jnp.float32),
            pltpu.VMEM((B, D), jnp.float32),
            pltpu.VMEM((1, D), jnp.float32),
            pltpu.VMEM_SHARED((N, D), jnp.float32),
            pltpu.SemaphoreType.DMA,
            pltpu.SemaphoreType.DMA,
            pltpu.SemaphoreType.DMA,
            pltpu.SemaphoreType.DMA,
            pltpu.SemaphoreType.DMA,
            pltpu.SemaphoreType.DMA,
            pltpu.SemaphoreType.DMA,
            pltpu.SemaphoreType.DMA,
        ],
    )
    return f(t0, t1, ivec, jvec, u2)


def _final_body(nodes_ref, h0_ref, h1_ref, w2_ref, out_ref):
    w2 = w2_ref[...]
    m = lax.dot_general(h0_ref[...], w2[:, :H // 2],
                        (((1,), (1,)), ((), ())),
                        preferred_element_type=jnp.float32)
    m += lax.dot_general(h1_ref[...], w2[:, H // 2:],
                         (((1,), (1,)), ((), ())),
                         preferred_element_type=jnp.float32)
    out_ref[...] = nodes_ref[...] + m


def _finalize(nodes, h0, h1, W2):
    nb = 10
    blk = N // nb
    return pl.pallas_call(
        _final_body,
        grid=(nb,),
        in_specs=[
            pl.BlockSpec((blk, D), lambda b: (b, 0)),
            pl.BlockSpec((blk, D), lambda b: (b, 0)),
            pl.BlockSpec((blk, D), lambda b: (b, 0)),
            pl.BlockSpec((D, H), lambda b: (0, 0)),
        ],
        out_specs=pl.BlockSpec((blk, D), lambda b: (b, 0)),
        out_shape=jax.ShapeDtypeStruct((N, D), jnp.float32),
    )(nodes, h0, h1, W2)


@jax.jit
def kernel(nodes, edges, W1, b1, W2, b2):
    table = _build_table(nodes, W1, b1)
    ivec = edges[:, 0].astype(jnp.int32)
    jvec = edges[:, 1].astype(jnp.int32)
    u = W2.T @ jnp.linalg.solve(W2 @ W2.T, b2)
    u2 = jnp.zeros((NC, 1, D), jnp.float32).at[:, 0, :].set(u.reshape(NC, D))
    acc = _sc_edge_accumulate(table[0], table[1], ivec, jvec, u2)
    return _finalize(nodes, acc[0], acc[1], W2)


# speedup vs baseline: 2.5516x; 2.5516x over previous
import jax
import jax.numpy as jnp
from jax import lax
from jax.experimental import pallas as pl
from jax.experimental.pallas import tpu as pltpu
from jax.experimental.pallas import tpu_sc as plsc

N = 10000
D = 128
H = 256
E = 320000

NC = 2
NS = 16
B = 40
EPT = E // NS
NBATCH = EPT // B
NP = NBATCH // 2
WB = 624


def _table_body(nodes_ref, w1_ref, b1_ref, out_ref):
    n = nodes_ref[...]
    w = w1_ref[...]
    pa = lax.dot_general(n, w[:, :D], (((1,), (1,)), ((), ())),
                         preferred_element_type=jnp.float32)
    pb = lax.dot_general(n, w[:, D:], (((1,), (1,)), ((), ())),
                         preferred_element_type=jnp.float32)
    out_ref[0, :, :D] = pa + b1_ref[0, 0:1, :]
    out_ref[0, :, D:] = pb


def _build_table(nodes, W1, b1):
    nb = 10
    blk = N // nb
    return pl.pallas_call(
        _table_body,
        grid=(NC, nb),
        in_specs=[
            pl.BlockSpec((blk, D), lambda c, b: (b, 0)),
            pl.BlockSpec((H // NC, 2 * D), lambda c, b: (c, 0)),
            pl.BlockSpec((1, 1, H // NC), lambda c, b: (c, 0, 0)),
        ],
        out_specs=pl.BlockSpec((1, blk, 2 * D), lambda c, b: (c, b, 0)),
        out_shape=jax.ShapeDtypeStruct((NC, N, 2 * D), jnp.float32),
    )(nodes, W1, b1.reshape(NC, 1, H // NC))


def _sc_edge_body(t0, t1, ivec, jvec, u2,
                  acc_out,
                  idx_ia, idx_ja, idx_ib, idx_jb,
                  rows_ia, rows_ja, rows_ib, rows_jb,
                  h_v, u_v,
                  acc_sh,
                  sga1, sga2, sgb1, sgb2, ssc, ssc2, si1, si2):
    c = lax.axis_index("c")
    s = lax.axis_index("s")

    def zrow(r, _):
        for v in range(D // 16):
            h_v[r, pl.ds(v * 16, 16)] = jnp.zeros((16,), jnp.float32)
        return 0
    lax.fori_loop(0, B, zrow, 0)
    for k in range(15):
        pltpu.sync_copy(h_v, acc_sh.at[pl.ds(s * (N // NS) + k * B, B)])
    pltpu.sync_copy(h_v.at[pl.ds(0, 25)],
                    acc_sh.at[pl.ds(s * (N // NS) + 15 * B, 25)])
    plsc.subcore_barrier()

    pltpu.sync_copy(u2.at[c], u_v)
    uv = [u_v[0, pl.ds(v * 16, 16)] for v in range(D // 16)]

    def load_idx(t, ii, jj):
        base = s * EPT + t * B
        cpi = pltpu.make_async_copy(ivec.at[pl.ds(base, B)], ii.at[0], si1)
        cpj = pltpu.make_async_copy(jvec.at[pl.ds(base, B)], jj.at[0], si2)
        cpi.start()
        cpj.start()
        cpi.wait()
        cpj.wait()

    NSPL = 5
    SB = B // NSPL

    def _gather_parts(tref, ii, jj, ri, rj, sg1, sg2):
        parts = []
        for k in range(NSPL):
            ksl = pl.ds(k * SB, SB)
            parts.append(pltpu.make_async_copy(
                tref.at[ii.at[0, ksl]], ri.at[ksl], sg1))
            parts.append(pltpu.make_async_copy(
                tref.at[jj.at[0, ksl]], rj.at[ksl], sg2))
        return parts

    def start_gathers(ii, jj, ri, rj, sg1, sg2):
        @pl.when(c == 0)
        def _():
            for cp in _gather_parts(t0, ii, jj, ri, rj, sg1, sg2):
                cp.start()

        @pl.when(c == 1)
        def _():
            for cp in _gather_parts(t1, ii, jj, ri, rj, sg1, sg2):
                cp.start()

    def wait_gathers(ii, jj, ri, rj, sg1, sg2):
        @pl.when(c == 0)
        def _():
            for cp in _gather_parts(t0, ii, jj, ri, rj, sg1, sg2):
                cp.wait()

        @pl.when(c == 1)
        def _():
            for cp in _gather_parts(t1, ii, jj, ri, rj, sg1, sg2):
                cp.wait()

    def compute_h(rx, ry, hbuf):
        @plsc.parallel_loop(0, B, 1, unroll=4)
        def _(b):
            for v in range(D // 16):
                sl = pl.ds(v * 16, 16)
                sh = pl.ds(D + v * 16, 16)
                hbuf[b, sl] = jnp.maximum(
                    rx[b, sl] + ry[b, sh], 0.0) + uv[v]

    def _scatter_parts(hbuf, idxr, sem):
        parts = []
        for k in range(NSPL):
            ksl = pl.ds(k * SB, SB)
            parts.append(pltpu.make_async_copy(
                hbuf.at[ksl], acc_sh.at[idxr.at[0, ksl]], sem))
        return parts

    def consume(ii, jj, ri, rj, sg1, sg2):
        wait_gathers(ii, jj, ri, rj, sg1, sg2)
        compute_h(ri, rj, h_v)
        p1 = _scatter_parts(h_v, ii, ssc)
        for cp in p1:
            cp.start(add=True)
        for cp in p1:
            cp.wait()
        compute_h(rj, ri, h_v)
        p2 = _scatter_parts(h_v, jj, ssc2)
        for cp in p2:
            cp.start(add=True)
        for cp in p2:
            cp.wait()

    load_idx(0, idx_ia, idx_ja)
    start_gathers(idx_ia, idx_ja, rows_ia, rows_ja, sga1, sga2)

    def pair_body(p, _):
        load_idx(2 * p + 1, idx_ib, idx_jb)
        start_gathers(idx_ib, idx_jb, rows_ib, rows_jb, sgb1, sgb2)
        consume(idx_ia, idx_ja, rows_ia, rows_ja, sga1, sga2)

        @pl.when(p < NP - 1)
        def _():
            load_idx(2 * p + 2, idx_ia, idx_ja)
            start_gathers(idx_ia, idx_ja, rows_ia, rows_ja, sga1, sga2)
        consume(idx_ib, idx_jb, rows_ib, rows_jb, sgb1, sgb2)
        return 0
    lax.fori_loop(0, NP, pair_body, 0)

    plsc.subcore_barrier()

    rsl = pl.ds(s * WB, WB)
    pltpu.sync_copy(acc_sh.at[rsl], acc_out.at[c, rsl])

    @pl.when(s == NS - 1)
    def _():
        tsl = pl.ds(NS * WB, N - NS * WB)
        pltpu.sync_copy(acc_sh.at[tsl], acc_out.at[c, tsl])


def _sc_edge_accumulate(t0, t1, ivec, jvec, u2):
    mesh = plsc.VectorSubcoreMesh(core_axis_name="c", subcore_axis_name="s",
                                  num_cores=NC, num_subcores=NS)
    f = pl.kernel(
        _sc_edge_body,
        out_type=jax.ShapeDtypeStruct((NC, N, D), jnp.float32),
        mesh=mesh,
        scratch_types=[
            pltpu.VMEM((1, B), jnp.int32),
            pltpu.VMEM((1, B), jnp.int32),
            pltpu.VMEM((1, B), jnp.int32),
            pltpu.VMEM((1, B), jnp.int32),
            pltpu.VMEM((B, 2 * D), jnp.float32),
            pltpu.VMEM((B, 2 * D), jnp.float32),
            pltpu.VMEM((B, 2 * D), jnp.float32),
            pltpu.VMEM((B, 2 * D), jnp.float32),
            pltpu.VMEM((B, D), jnp.float32),
            pltpu.VMEM((1, D), jnp.float32),
            pltpu.VMEM_SHARED((N, D), jnp.float32),
            pltpu.SemaphoreType.DMA,
            pltpu.SemaphoreType.DMA,
            pltpu.SemaphoreType.DMA,
            pltpu.SemaphoreType.DMA,
            pltpu.SemaphoreType.DMA,
            pltpu.SemaphoreType.DMA,
            pltpu.SemaphoreType.DMA,
            pltpu.SemaphoreType.DMA,
        ],
    )
    return f(t0, t1, ivec, jvec, u2)


def _final_body(nodes_ref, h0_ref, h1_ref, w2_ref, out_ref):
    w2 = w2_ref[...]
    m = lax.dot_general(h0_ref[...], w2[:, :H // 2],
                        (((1,), (1,)), ((), ())),
                        preferred_element_type=jnp.float32)
    m += lax.dot_general(h1_ref[...], w2[:, H // 2:],
                         (((1,), (1,)), ((), ())),
                         preferred_element_type=jnp.float32)
    out_ref[...] = nodes_ref[...] + m


def _finalize(nodes, h0, h1, W2):
    nb = 10
    blk = N // nb
    return pl.pallas_call(
        _final_body,
        grid=(nb,),
        in_specs=[
            pl.BlockSpec((blk, D), lambda b: (b, 0)),
            pl.BlockSpec((blk, D), lambda b: (b, 0)),
            pl.BlockSpec((blk, D), lambda b: (b, 0)),
            pl.BlockSpec((D, H), lambda b: (0, 0)),
        ],
        out_specs=pl.BlockSpec((blk, D), lambda b: (b, 0)),
        out_shape=jax.ShapeDtypeStruct((N, D), jnp.float32),
    )(nodes, h0, h1, W2)


@jax.jit
def kernel(nodes, edges, W1, b1, W2, b2):
    table = _build_table(nodes, W1, b1)
    ivec = edges[:, 0].astype(jnp.int32)
    jvec = edges[:, 1].astype(jnp.int32)
    u = W2.T @ jnp.linalg.solve(W2 @ W2.T, b2)
    u2 = jnp.zeros((NC, 1, D), jnp.float32).at[:, 0, :].set(u.reshape(NC, D))
    acc = _sc_edge_accumulate(table[0], table[1], ivec, jvec, u2)
    return _finalize(nodes, acc[0], acc[1], W2)
